# Initial kernel scaffold; baseline (speedup 1.0000x reference)
import jax
import jax.numpy as jnp
from jax.experimental import pallas as pl


def kernel(h, Ww, bw, Wn, bn, eps):
    raise NotImplementedError("write your pallas kernel here")


# speedup vs baseline: 2.2650x; 2.2650x over previous
import functools

import jax
import jax.numpy as jnp
from jax.experimental import pallas as pl
from jax.experimental.pallas import tpu as pltpu

N = 8192
D = 2048
E = 16
BN = 512


def _router_block(h_ref, wt_ref, b_ref, eps_ref, probs_ref, ix_ref, full_ref):
    z = jnp.dot(h_ref[...], wt_ref[...], preferred_element_type=jnp.float32)
    z = z + b_ref[...]
    logits = z[:, :E]
    noise = eps_ref[...] * jax.nn.softplus(z[:, E:])
    noisy = logits + noise

    m1 = jnp.max(noisy, axis=-1, keepdims=True)
    e = jnp.exp(noisy - m1)
    full_ref[...] = e / jnp.sum(e, axis=-1, keepdims=True)

    lane = jax.lax.broadcasted_iota(jnp.int32, (BN, E), 1)
    ix1 = jnp.min(jnp.where(noisy == m1, lane, E), axis=-1, keepdims=True)
    masked = jnp.where(lane == ix1, -jnp.inf, noisy)
    m2 = jnp.max(masked, axis=-1, keepdims=True)
    ix2 = jnp.min(jnp.where(masked == m2, lane, E), axis=-1, keepdims=True)

    t = jnp.exp(m2 - m1)
    denom = 1.0 + t
    probs_ref[...] = jnp.where(
        lane == ix1, 1.0 / denom, jnp.where(lane == ix2, t / denom, 0.0)
    )
    ix_ref[...] = jnp.concatenate([ix1, ix2], axis=-1)


@jax.jit
def kernel(h, Ww, bw, Wn, bn, eps):
    wt = jnp.concatenate([Ww, Wn], axis=0).T
    b = jnp.concatenate([bw, bn]).reshape(1, 2 * E)
    grid = (N // BN,)
    probs, ix, full = pl.pallas_call(
        _router_block,
        grid=grid,
        in_specs=[
            pl.BlockSpec((BN, D), lambda i: (i, 0)),
            pl.BlockSpec((D, 2 * E), lambda i: (0, 0)),
            pl.BlockSpec((1, 2 * E), lambda i: (0, 0)),
            pl.BlockSpec((BN, E), lambda i: (i, 0)),
        ],
        out_specs=[
            pl.BlockSpec((BN, E), lambda i: (i, 0)),
            pl.BlockSpec((BN, 2), lambda i: (i, 0)),
            pl.BlockSpec((BN, E), lambda i: (i, 0)),
        ],
        out_shape=[
            jax.ShapeDtypeStruct((N, E), jnp.float32),
            jax.ShapeDtypeStruct((N, 2), jnp.int32),
            jax.ShapeDtypeStruct((N, E), jnp.float32),
        ],
    )(h, wt, b, eps)
    return probs, ix, full


# speedup vs baseline: 2.3968x; 1.0582x over previous
import functools

import jax
import jax.numpy as jnp
from jax.experimental import pallas as pl
from jax.experimental.pallas import tpu as pltpu

N = 8192
D = 2048
E = 16
BN = 1024


def _router_block(h_ref, wt_ref, b_ref, eps_ref, probs_ref, ix_ref, full_ref):
    z = jnp.dot(h_ref[...], wt_ref[...], preferred_element_type=jnp.float32)
    z = z + b_ref[...]
    logits = z[:, :E]
    noise = eps_ref[...] * jax.nn.softplus(z[:, E:])
    noisy = logits + noise

    m1 = jnp.max(noisy, axis=-1, keepdims=True)
    e = jnp.exp(noisy - m1)
    full_ref[...] = e / jnp.sum(e, axis=-1, keepdims=True)

    lane = jax.lax.broadcasted_iota(jnp.int32, (BN, E), 1)
    ix1 = jnp.min(jnp.where(noisy == m1, lane, E), axis=-1, keepdims=True)
    masked = jnp.where(lane == ix1, -jnp.inf, noisy)
    m2 = jnp.max(masked, axis=-1, keepdims=True)
    ix2 = jnp.min(jnp.where(masked == m2, lane, E), axis=-1, keepdims=True)

    t = jnp.exp(m2 - m1)
    denom = 1.0 + t
    probs_ref[...] = jnp.where(
        lane == ix1, 1.0 / denom, jnp.where(lane == ix2, t / denom, 0.0)
    )
    ix_ref[...] = jnp.concatenate([ix1, ix2], axis=-1)


@jax.jit
def kernel(h, Ww, bw, Wn, bn, eps):
    wt = jnp.concatenate([Ww, Wn], axis=0).T
    b = jnp.concatenate([bw, bn]).reshape(1, 2 * E)
    grid = (N // BN,)
    probs, ix, full = pl.pallas_call(
        _router_block,
        grid=grid,
        in_specs=[
            pl.BlockSpec((BN, D), lambda i: (i, 0)),
            pl.BlockSpec((D, 2 * E), lambda i: (0, 0)),
            pl.BlockSpec((1, 2 * E), lambda i: (0, 0)),
            pl.BlockSpec((BN, E), lambda i: (i, 0)),
        ],
        out_specs=[
            pl.BlockSpec((BN, E), lambda i: (i, 0)),
            pl.BlockSpec((BN, 2), lambda i: (i, 0)),
            pl.BlockSpec((BN, E), lambda i: (i, 0)),
        ],
        out_shape=[
            jax.ShapeDtypeStruct((N, E), jnp.float32),
            jax.ShapeDtypeStruct((N, 2), jnp.int32),
            jax.ShapeDtypeStruct((N, E), jnp.float32),
        ],
    )(h, wt, b, eps)
    return probs, ix, full
